# Initial kernel scaffold; baseline (speedup 1.0000x reference)
import jax
import jax.numpy as jnp
from jax.experimental import pallas as pl


def kernel(x):
    raise NotImplementedError("write your pallas kernel here")


# speedup vs baseline: 3.0918x; 3.0918x over previous
import functools

import jax
import jax.numpy as jnp
from jax.experimental import pallas as pl

L = 4096
BLK = 256


def _cumsum_kernel(x_ref, o_ref):
    x = x_ref[0]
    k = 1
    while k < L:
        x = x + jnp.concatenate(
            [jnp.zeros((k, x.shape[1]), x.dtype), x[:-k]], axis=0
        )
        k *= 2
    o_ref[0] = x


@jax.jit
def kernel(x):
    b, l, d = x.shape
    grid = (b, d // BLK)
    return pl.pallas_call(
        _cumsum_kernel,
        grid=grid,
        in_specs=[pl.BlockSpec((1, l, BLK), lambda i, j: (i, 0, j))],
        out_specs=pl.BlockSpec((1, l, BLK), lambda i, j: (i, 0, j)),
        out_shape=jax.ShapeDtypeStruct(x.shape, x.dtype),
    )(x)


# speedup vs baseline: 3.1017x; 1.0032x over previous
import functools

import jax
import jax.numpy as jnp
from jax.experimental import pallas as pl
from jax.experimental.pallas import tpu as pltpu

L = 4096
BLK = 512


def _cumsum_kernel(x_ref, o_ref):
    x = x_ref[0]
    k = 1
    while k < L:
        x = x + jnp.concatenate(
            [jnp.zeros((k, x.shape[1]), x.dtype), x[:-k]], axis=0
        )
        k *= 2
    o_ref[0] = x


@jax.jit
def kernel(x):
    b, l, d = x.shape
    grid = (b, d // BLK)
    return pl.pallas_call(
        _cumsum_kernel,
        grid=grid,
        in_specs=[pl.BlockSpec((1, l, BLK), lambda i, j: (i, 0, j))],
        out_specs=pl.BlockSpec((1, l, BLK), lambda i, j: (i, 0, j)),
        out_shape=jax.ShapeDtypeStruct(x.shape, x.dtype),
        compiler_params=pltpu.CompilerParams(
            dimension_semantics=("parallel", "parallel"),
        ),
    )(x)
